# baseline (speedup 1.0000x reference)
import functools

import jax
import jax.numpy as jnp
from jax import lax
from jax.experimental import pallas as pl
from jax.experimental.pallas import tpu as pltpu
from jax.experimental.pallas import tpu_sc as plsc

VOCAB = 50257
D = 768
S = 1024
B = 4
TOK = B * S
NC, NS = 2, 16
NW = NC * NS
PW = S // NW
NVEC = D // 16
NBUF = 4

_mesh = plsc.VectorSubcoreMesh(core_axis_name="c", subcore_axis_name="s")


@functools.partial(
    pl.kernel,
    mesh=_mesh,
    out_type=jax.ShapeDtypeStruct((TOK, D), jnp.float32),
    scratch_types=[
        pltpu.VMEM((B, PW), jnp.int32),
        pltpu.VMEM((PW, D), jnp.float32),
        pltpu.VMEM((PW, D), jnp.float32),
        pltpu.VMEM((PW, D), jnp.float32),
        pltpu.VMEM((PW, D), jnp.float32),
        pltpu.VMEM((PW, D), jnp.float32),
        pltpu.SemaphoreType.DMA,
        pltpu.SemaphoreType.DMA,
        pltpu.SemaphoreType.DMA,
        pltpu.SemaphoreType.DMA,
        pltpu.SemaphoreType.DMA,
        pltpu.SemaphoreType.DMA,
        pltpu.SemaphoreType.DMA,
        pltpu.SemaphoreType.DMA,
        pltpu.SemaphoreType.DMA,
        pltpu.SemaphoreType.DMA,
    ],
)
def _embed(ids_hbm, wte_hbm, wpe_hbm, out_hbm,
           idx_v, r0, r1, r2, r3, pos_v,
           sg0, sg1, sg2, sg3, ss0, ss1, ss2, ss3, spos, sidx):
    rows = (r0, r1, r2, r3)
    sg = (sg0, sg1, sg2, sg3)
    ss = (ss0, ss1, ss2, ss3)
    wid = lax.axis_index("s") * NC + lax.axis_index("c")
    p0 = wid * PW

    pre = pltpu.async_copy(wpe_hbm.at[pl.ds(p0, PW)], pos_v, spos)
    id_h = [pltpu.async_copy(ids_hbm.at[bb, pl.ds(p0, PW)], idx_v.at[bb],
                             sidx)
            for bb in range(B)]

    def start_gather(ch):
        id_h[ch].wait()
        return pltpu.async_copy(
            wte_hbm.at[idx_v.at[ch]], rows[ch % NBUF], sg[ch % NBUF])

    inflight = {ch: start_gather(ch) for ch in range(B)}
    pre.wait()

    store_h = [None, None, None, None]
    for ch in range(B):
        b = ch % NBUF
        inflight.pop(ch).wait()

        def add_row(r, carry):
            for j in range(NVEC):
                plsc.addupdate(rows[b].at[r, pl.ds(j * 16, 16)],
                               pos_v[r, pl.ds(j * 16, 16)])
            return carry

        lax.fori_loop(0, PW, add_row, 0)
        store_h[b] = pltpu.async_copy(
            rows[b], out_hbm.at[pl.ds(ch * S + p0, PW)], ss[b])
    for h in store_h:
        if h is not None:
            h.wait()


def kernel(input_ids, wte, wpe):
    out = _embed(input_ids.astype(jnp.int32), wte, wpe)
    return out.reshape(input_ids.shape + (wpe.shape[1],))


# speedup vs baseline: 1.0208x; 1.0208x over previous
import functools

import jax
import jax.numpy as jnp
from jax import lax
from jax.experimental import pallas as pl
from jax.experimental.pallas import tpu as pltpu
from jax.experimental.pallas import tpu_sc as plsc

VOCAB = 50257
D = 768
S = 1024
B = 4
TOK = B * S
NC, NS = 2, 16
NW = NC * NS
PW = S // NW
NVEC = D // 16
NBUF = 3

_mesh = plsc.VectorSubcoreMesh(core_axis_name="c", subcore_axis_name="s")


@functools.partial(
    pl.kernel,
    mesh=_mesh,
    out_type=jax.ShapeDtypeStruct((TOK, D), jnp.float32),
    scratch_types=[
        pltpu.VMEM((B, PW), jnp.int32),
        pltpu.VMEM((PW, D), jnp.float32),
        pltpu.VMEM((PW, D), jnp.float32),
        pltpu.VMEM((PW, D), jnp.float32),
        pltpu.VMEM((PW, D), jnp.float32),
        pltpu.SemaphoreType.DMA,
        pltpu.SemaphoreType.DMA,
        pltpu.SemaphoreType.DMA,
        pltpu.SemaphoreType.DMA,
        pltpu.SemaphoreType.DMA,
        pltpu.SemaphoreType.DMA,
        pltpu.SemaphoreType.DMA,
        pltpu.SemaphoreType.DMA,
    ],
)
def _embed(ids_hbm, wte_hbm, wpe_hbm, out_hbm,
           idx_v, r0, r1, r2, pos_v,
           sg0, sg1, sg2, ss0, ss1, ss2, spos, sidx):
    rows = (r0, r1, r2)
    sg = (sg0, sg1, sg2)
    ss = (ss0, ss1, ss2)
    wid = lax.axis_index("s") * NC + lax.axis_index("c")
    p0 = wid * PW

    pre = pltpu.async_copy(wpe_hbm.at[pl.ds(p0, PW)], pos_v, spos)
    id_h = [pltpu.async_copy(ids_hbm.at[bb, pl.ds(p0, PW)], idx_v.at[bb],
                             sidx)
            for bb in range(B)]

    def start_gather(ch):
        id_h[ch].wait()
        return pltpu.async_copy(
            wte_hbm.at[idx_v.at[ch]], rows[ch % NBUF], sg[ch % NBUF])

    inflight = {0: start_gather(0), 1: start_gather(1)}
    pre.wait()

    store_h = [None, None, None]
    for ch in range(B):
        b = ch % NBUF
        if ch + 2 < B:
            nb = (ch + 2) % NBUF
            if store_h[nb] is not None:
                store_h[nb].wait()
                store_h[nb] = None
            inflight[ch + 2] = start_gather(ch + 2)
        inflight.pop(ch).wait()

        def add_row(r, carry):
            for j in range(NVEC):
                plsc.addupdate(rows[b].at[r, pl.ds(j * 16, 16)],
                               pos_v[r, pl.ds(j * 16, 16)])
            return carry

        lax.fori_loop(0, PW, add_row, 0)
        store_h[b] = pltpu.async_copy(
            rows[b], out_hbm.at[pl.ds(ch * S + p0, PW)], ss[b])
    for h in store_h:
        if h is not None:
            h.wait()


def kernel(input_ids, wte, wpe):
    out = _embed(input_ids.astype(jnp.int32), wte, wpe)
    return out.reshape(input_ids.shape + (wpe.shape[1],))
